# baseline (speedup 1.0000x reference)
import functools

import jax
import jax.numpy as jnp
from jax import lax
from jax.experimental import pallas as pl
from jax.experimental.pallas import tpu as pltpu
from jax.experimental.pallas import tpu_sc as plsc

_N = 10000
_E = 320000
_D = 128
_NP = 10112
_RPT = _NP // 16
_CK = 128
_CH = 80
_TILES = 32
_EPAD = _TILES * _CH * _CK
_NB = 2
_DNB = 4
_QC = 40


def _sc_mesh():
    return plsc.VectorSubcoreMesh(
        core_axis_name="c", subcore_axis_name="s", num_cores=2, num_subcores=16
    )


def _deg_body(dst_hbm, init_hbm, ones_hbm, out_hbm, dst_v, ones_v,
              d0, d1, d2, d3, acc_sh):
    c = lax.axis_index("c")
    s = lax.axis_index("s")
    wid = c * 16 + s
    sems = (d0, d1, d2, d3)
    pltpu.sync_copy(dst_hbm.at[wid], dst_v)
    pltpu.sync_copy(ones_hbm, ones_v)
    pltpu.sync_copy(init_hbm.at[pl.ds(s * _RPT, _RPT)],
                    acc_sh.at[pl.ds(s * _RPT, _RPT)])
    plsc.subcore_barrier()

    @pl.loop(0, _CH, step=_DNB)
    def _chunk(j):
        for b in range(_DNB):
            jb = j + b
            pltpu.async_copy(ones_v, acc_sh.at[dst_v.at[jb]], sems[b],
                             add=True)

            @pl.when(j > 0)
            def _():
                pltpu.make_async_copy(
                    ones_v, acc_sh.at[dst_v.at[jb - _DNB]], sems[b]).wait()

    for b in range(_DNB):
        pltpu.make_async_copy(
            ones_v, acc_sh.at[dst_v.at[_CH - _DNB + b]], sems[b]).wait()

    plsc.subcore_barrier()
    pltpu.sync_copy(acc_sh.at[pl.ds(s * _RPT, _RPT)],
                    out_hbm.at[c, pl.ds(s * _RPT, _RPT)])


def _deg_call(dst_idx):
    k = pl.kernel(
        _deg_body,
        out_type=jax.ShapeDtypeStruct((2, _NP, 8), jnp.float32),
        mesh=_sc_mesh(),
        scratch_types=[
            pltpu.VMEM((_CH, _CK), jnp.int32),
            pltpu.VMEM((_CK, 8), jnp.float32),
            pltpu.SemaphoreType.DMA,
            pltpu.SemaphoreType.DMA,
            pltpu.SemaphoreType.DMA,
            pltpu.SemaphoreType.DMA,
            pltpu.VMEM_SHARED((_NP, 8), jnp.float32),
        ],
    )
    init = jnp.full((_NP, 8), 0.5, dtype=jnp.float32)
    ones = jnp.ones((_CK, 8), dtype=jnp.float32)
    parts = k(dst_idx, init, ones)
    return parts[0] + parts[1]


def _spmm_body(z_hbm, src_hbm, dst_hbm, zeros_hbm, out_hbm,
               src_v, dst_v,
               r0, r1,
               g0, g1,
               t0, t1, acc_sh):
    c = lax.axis_index("c")
    s = lax.axis_index("s")
    wid = c * 16 + s
    rows = (r0, r1)
    gsem = (g0, g1)
    ssem = (t0, t1)
    pltpu.sync_copy(zeros_hbm, acc_sh.at[pl.ds(s * _RPT, _RPT)])
    plsc.subcore_barrier()

    for q in range(_CH // _QC):
        pltpu.sync_copy(src_hbm.at[wid, pl.ds(q * _QC, _QC)], src_v)
        pltpu.sync_copy(dst_hbm.at[wid, pl.ds(q * _QC, _QC)], dst_v)
        pltpu.async_copy(z_hbm.at[src_v.at[0]], rows[0], gsem[0])

        @pl.loop(0, _QC, step=_NB)
        def _chunks(j):
            for b in range(_NB):
                jb = j + b
                b2 = (b + 1) % _NB
                pltpu.make_async_copy(
                    z_hbm.at[src_v.at[jb]], rows[b], gsem[b]).wait()
                pltpu.async_copy(rows[b], acc_sh.at[dst_v.at[jb]], ssem[b],
                                 add=True)
                if b == 0:
                    @pl.when(j > 0)
                    def _():
                        pltpu.make_async_copy(
                            rows[b2], acc_sh.at[dst_v.at[jb - 1]],
                            ssem[b2]).wait()
                        pltpu.async_copy(
                            z_hbm.at[src_v.at[jb + 1]], rows[b2], gsem[b2])

                    @pl.when(j == 0)
                    def _():
                        pltpu.async_copy(
                            z_hbm.at[src_v.at[1]], rows[b2], gsem[b2])
                else:
                    pltpu.make_async_copy(
                        rows[b2], acc_sh.at[dst_v.at[jb - 1]], ssem[b2]).wait()

                    @pl.when(jb + 1 < _QC)
                    def _():
                        pltpu.async_copy(
                            z_hbm.at[src_v.at[jb + 1]], rows[b2], gsem[b2])

        pltpu.make_async_copy(
            rows[(_QC - 1) % _NB], acc_sh.at[dst_v.at[_QC - 1]],
            ssem[(_QC - 1) % _NB]).wait()

    plsc.subcore_barrier()
    pltpu.sync_copy(acc_sh.at[pl.ds(s * _RPT, _RPT)],
                    out_hbm.at[c, pl.ds(s * _RPT, _RPT)])


def _spmm_call(z, src_idx, dst_idx, zeros_rpt):
    k = pl.kernel(
        _spmm_body,
        out_type=jax.ShapeDtypeStruct((2, _NP, _D), jnp.float32),
        mesh=_sc_mesh(),
        scratch_types=[
            pltpu.VMEM((_QC, _CK), jnp.int32),
            pltpu.VMEM((_QC, _CK), jnp.int32),
            pltpu.VMEM((_CK, _D), jnp.float32),
            pltpu.VMEM((_CK, _D), jnp.float32),
            pltpu.SemaphoreType.DMA,
            pltpu.SemaphoreType.DMA,
            pltpu.SemaphoreType.DMA,
            pltpu.SemaphoreType.DMA,
            pltpu.VMEM_SHARED((_NP, _D), jnp.float32),
        ],
    )
    return k(z, src_idx, dst_idx, zeros_rpt)


_R = 128
_G = _NP // _R


def _dis(deg_ref):
    return lax.rsqrt(deg_ref[:, 0:1])


def _mm_first_body(x_ref, w_ref, deg_ref, o_ref):
    o_ref[...] = (
        jnp.dot(x_ref[...], w_ref[...], preferred_element_type=jnp.float32)
        * _dis(deg_ref)
    )


def _mm_first(x, w, deg8):
    return pl.pallas_call(
        _mm_first_body,
        grid=(_G,),
        in_specs=[
            pl.BlockSpec((_R, _D), lambda i: (i, 0)),
            pl.BlockSpec((_D, _D), lambda i: (0, 0)),
            pl.BlockSpec((_R, 8), lambda i: (i, 0)),
        ],
        out_specs=pl.BlockSpec((_R, _D), lambda i: (i, 0)),
        out_shape=jax.ShapeDtypeStruct((_NP, _D), jnp.float32),
    )(x, w, deg8)


def _mm_mid_body(acc_ref, z_ref, deg_ref, b_ref, w_ref, o_ref):
    dis = _dis(deg_ref)
    h = jnp.maximum((acc_ref[0] + acc_ref[1] + z_ref[...]) * dis + b_ref[...], 0.0)
    o_ref[...] = jnp.dot(h, w_ref[...], preferred_element_type=jnp.float32) * dis


def _mm_mid(acc, z, deg8, b, w):
    return pl.pallas_call(
        _mm_mid_body,
        grid=(_G,),
        in_specs=[
            pl.BlockSpec((2, _R, _D), lambda i: (0, i, 0)),
            pl.BlockSpec((_R, _D), lambda i: (i, 0)),
            pl.BlockSpec((_R, 8), lambda i: (i, 0)),
            pl.BlockSpec((1, _D), lambda i: (0, 0)),
            pl.BlockSpec((_D, _D), lambda i: (0, 0)),
        ],
        out_specs=pl.BlockSpec((_R, _D), lambda i: (i, 0)),
        out_shape=jax.ShapeDtypeStruct((_NP, _D), jnp.float32),
    )(acc, z, deg8, b, w)


def _final_body(acc_ref, z_ref, deg_ref, b_ref, o_ref):
    o_ref[...] = (
        (acc_ref[0] + acc_ref[1] + z_ref[...]) * _dis(deg_ref) + b_ref[...]
    )


def _final(acc, z, deg8, b):
    return pl.pallas_call(
        _final_body,
        grid=(_G,),
        in_specs=[
            pl.BlockSpec((2, _R, _D), lambda i: (0, i, 0)),
            pl.BlockSpec((_R, _D), lambda i: (i, 0)),
            pl.BlockSpec((_R, 8), lambda i: (i, 0)),
            pl.BlockSpec((1, _D), lambda i: (0, 0)),
        ],
        out_specs=pl.BlockSpec((_R, _D), lambda i: (i, 0)),
        out_shape=jax.ShapeDtypeStruct((_NP, _D), jnp.float32),
    )(acc, z, deg8, b)


def _prep_edges(edge_index):
    order = jnp.argsort(edge_index[1])
    pad = jnp.full((_EPAD - _E,), _N, dtype=jnp.int32)
    src = jnp.concatenate([edge_index[0][order], pad]).reshape(
        _TILES, _CH, _CK)
    dst = jnp.concatenate([edge_index[1][order], pad]).reshape(
        _TILES, _CH, _CK)
    return src, dst


def _branch(x, edge_index, params, zeros_rpt):
    w1, b1, w2, b2, w3, b3, w4, b4 = params
    src, dst = _prep_edges(edge_index)
    xp = jnp.pad(x, ((0, _NP - _N), (0, 0)))
    deg8 = _deg_call(dst)
    z = _mm_first(xp, w1, deg8)
    ws = (w2, w3, w4)
    bs = (b1.reshape(1, _D), b2.reshape(1, _D), b3.reshape(1, _D),
          b4.reshape(1, _D))
    for layer in range(3):
        acc = _spmm_call(z, src, dst, zeros_rpt)
        z = _mm_mid(acc, z, deg8, bs[layer], ws[layer])
    acc = _spmm_call(z, src, dst, zeros_rpt)
    out = _final(acc, z, deg8, bs[3])
    return out[:_N]


def kernel(x_data_matrix, x_edge_index, y_data_matrix, y_edge_index,
           Wx1, bx1, Wx2, bx2, Wx3, bx3, Wx4, bx4,
           Wy1, by1, Wy2, by2, Wy3, by3, Wy4, by4):
    zeros_rpt = jnp.zeros((_RPT, _D), dtype=jnp.float32)
    xo = _branch(x_data_matrix, x_edge_index,
                 (Wx1, bx1, Wx2, bx2, Wx3, bx3, Wx4, bx4), zeros_rpt)
    yo = _branch(y_data_matrix, y_edge_index,
                 (Wy1, by1, Wy2, by2, Wy3, by3, Wy4, by4), zeros_rpt)
    return (xo, yo)


# speedup vs baseline: 1.3219x; 1.3219x over previous
import functools

import jax
import jax.numpy as jnp
from jax import lax
from jax.experimental import pallas as pl
from jax.experimental.pallas import tpu as pltpu
from jax.experimental.pallas import tpu_sc as plsc

_N = 10000
_E = 320000
_D = 128
_NP = 10112
_RPT = _NP // 16
_CK = 128
_CH = 80
_TILES = 32
_EPAD = _TILES * _CH * _CK
_NB = 2
_QC = 40


def _sc_mesh():
    return plsc.VectorSubcoreMesh(
        core_axis_name="c", subcore_axis_name="s", num_cores=2, num_subcores=16
    )


def _deg_body(dst_hbm, init_hbm, ones_hbm, out_hbm, dst_v, ones_v, acc_sh):
    c = lax.axis_index("c")
    s = lax.axis_index("s")
    wid = c * 16 + s
    pltpu.sync_copy(dst_hbm.at[wid], dst_v)
    pltpu.sync_copy(ones_hbm, ones_v)
    pltpu.sync_copy(init_hbm.at[pl.ds(s * _RPT, _RPT)],
                    acc_sh.at[pl.ds(s * _RPT, _RPT)])
    plsc.subcore_barrier()

    @pl.loop(0, _CH)
    def _chunk(j):
        pltpu.sync_copy(ones_v, acc_sh.at[dst_v.at[j]], add=True)

    plsc.subcore_barrier()
    pltpu.sync_copy(acc_sh.at[pl.ds(s * _RPT, _RPT)],
                    out_hbm.at[c, pl.ds(s * _RPT, _RPT)])


def _deg_call(dst_idx):
    k = pl.kernel(
        _deg_body,
        out_type=jax.ShapeDtypeStruct((2, _NP, 8), jnp.float32),
        mesh=_sc_mesh(),
        scratch_types=[
            pltpu.VMEM((_CH, _CK), jnp.int32),
            pltpu.VMEM((_CK, 8), jnp.float32),
            pltpu.VMEM_SHARED((_NP, 8), jnp.float32),
        ],
    )
    init = jnp.full((_NP, 8), 0.5, dtype=jnp.float32)
    ones = jnp.ones((_CK, 8), dtype=jnp.float32)
    parts = k(dst_idx, init, ones)
    return parts[0] + parts[1]


def _spmm_body(z_hbm, src_hbm, dst_hbm, zeros_hbm, out_hbm,
               src_v, dst_v,
               r0, r1,
               g0, g1, acc_sh):
    c = lax.axis_index("c")
    s = lax.axis_index("s")
    wid = c * 16 + s
    rows = (r0, r1)
    gsem = (g0, g1)
    pltpu.sync_copy(zeros_hbm, acc_sh.at[pl.ds(s * _RPT, _RPT)])
    plsc.subcore_barrier()

    for q in range(_CH // _QC):
        pltpu.sync_copy(src_hbm.at[wid, pl.ds(q * _QC, _QC)], src_v)
        pltpu.sync_copy(dst_hbm.at[wid, pl.ds(q * _QC, _QC)], dst_v)
        for b in range(_NB):
            pltpu.async_copy(z_hbm.at[src_v.at[b]], rows[b], gsem[b])

        @pl.loop(0, _QC, step=_NB)
        def _chunks(j):
            for b in range(_NB):
                jb = j + b
                pltpu.make_async_copy(
                    z_hbm.at[src_v.at[jb]], rows[b], gsem[b]).wait()
                pltpu.sync_copy(rows[b], acc_sh.at[dst_v.at[jb]], add=True)

                @pl.when(jb + _NB < _QC)
                def _():
                    pltpu.async_copy(
                        z_hbm.at[src_v.at[jb + _NB]], rows[b], gsem[b])

    plsc.subcore_barrier()
    pltpu.sync_copy(acc_sh.at[pl.ds(s * _RPT, _RPT)],
                    out_hbm.at[c, pl.ds(s * _RPT, _RPT)])


def _spmm_call(z, src_idx, dst_idx, zeros_rpt):
    k = pl.kernel(
        _spmm_body,
        out_type=jax.ShapeDtypeStruct((2, _NP, _D), jnp.float32),
        mesh=_sc_mesh(),
        scratch_types=[
            pltpu.VMEM((_QC, _CK), jnp.int32),
            pltpu.VMEM((_QC, _CK), jnp.int32),
            pltpu.VMEM((_CK, _D), jnp.float32),
            pltpu.VMEM((_CK, _D), jnp.float32),
            pltpu.SemaphoreType.DMA,
            pltpu.SemaphoreType.DMA,
            pltpu.VMEM_SHARED((_NP, _D), jnp.float32),
        ],
    )
    return k(z, src_idx, dst_idx, zeros_rpt)


_R = 128
_G = _NP // _R


def _dis(deg_ref):
    return lax.rsqrt(deg_ref[:, 0:1])


def _mm_first_body(x_ref, w_ref, deg_ref, o_ref):
    o_ref[...] = (
        jnp.dot(x_ref[...], w_ref[...], preferred_element_type=jnp.float32)
        * _dis(deg_ref)
    )


def _mm_first(x, w, deg8):
    return pl.pallas_call(
        _mm_first_body,
        grid=(_G,),
        in_specs=[
            pl.BlockSpec((_R, _D), lambda i: (i, 0)),
            pl.BlockSpec((_D, _D), lambda i: (0, 0)),
            pl.BlockSpec((_R, 8), lambda i: (i, 0)),
        ],
        out_specs=pl.BlockSpec((_R, _D), lambda i: (i, 0)),
        out_shape=jax.ShapeDtypeStruct((_NP, _D), jnp.float32),
    )(x, w, deg8)


def _mm_mid_body(acc_ref, z_ref, deg_ref, b_ref, w_ref, o_ref):
    dis = _dis(deg_ref)
    h = jnp.maximum((acc_ref[0] + acc_ref[1] + z_ref[...]) * dis + b_ref[...], 0.0)
    o_ref[...] = jnp.dot(h, w_ref[...], preferred_element_type=jnp.float32) * dis


def _mm_mid(acc, z, deg8, b, w):
    return pl.pallas_call(
        _mm_mid_body,
        grid=(_G,),
        in_specs=[
            pl.BlockSpec((2, _R, _D), lambda i: (0, i, 0)),
            pl.BlockSpec((_R, _D), lambda i: (i, 0)),
            pl.BlockSpec((_R, 8), lambda i: (i, 0)),
            pl.BlockSpec((1, _D), lambda i: (0, 0)),
            pl.BlockSpec((_D, _D), lambda i: (0, 0)),
        ],
        out_specs=pl.BlockSpec((_R, _D), lambda i: (i, 0)),
        out_shape=jax.ShapeDtypeStruct((_NP, _D), jnp.float32),
    )(acc, z, deg8, b, w)


def _final_body(acc_ref, z_ref, deg_ref, b_ref, o_ref):
    o_ref[...] = (
        (acc_ref[0] + acc_ref[1] + z_ref[...]) * _dis(deg_ref) + b_ref[...]
    )


def _final(acc, z, deg8, b):
    return pl.pallas_call(
        _final_body,
        grid=(_G,),
        in_specs=[
            pl.BlockSpec((2, _R, _D), lambda i: (0, i, 0)),
            pl.BlockSpec((_R, _D), lambda i: (i, 0)),
            pl.BlockSpec((_R, 8), lambda i: (i, 0)),
            pl.BlockSpec((1, _D), lambda i: (0, 0)),
        ],
        out_specs=pl.BlockSpec((_R, _D), lambda i: (i, 0)),
        out_shape=jax.ShapeDtypeStruct((_NP, _D), jnp.float32),
    )(acc, z, deg8, b)


def _prep_edges(edge_index):
    pad = jnp.full((_EPAD - _E,), _N, dtype=jnp.int32)
    src = jnp.concatenate([edge_index[0], pad]).reshape(_TILES, _CH, _CK)
    dst = jnp.concatenate([edge_index[1], pad]).reshape(_TILES, _CH, _CK)
    return src, dst


def _branch(x, edge_index, params, zeros_rpt):
    w1, b1, w2, b2, w3, b3, w4, b4 = params
    src, dst = _prep_edges(edge_index)
    xp = jnp.pad(x, ((0, _NP - _N), (0, 0)))
    deg8 = _deg_call(dst)
    z = _mm_first(xp, w1, deg8)
    ws = (w2, w3, w4)
    bs = (b1.reshape(1, _D), b2.reshape(1, _D), b3.reshape(1, _D),
          b4.reshape(1, _D))
    for layer in range(3):
        acc = _spmm_call(z, src, dst, zeros_rpt)
        z = _mm_mid(acc, z, deg8, bs[layer], ws[layer])
    acc = _spmm_call(z, src, dst, zeros_rpt)
    out = _final(acc, z, deg8, bs[3])
    return out[:_N]


def kernel(x_data_matrix, x_edge_index, y_data_matrix, y_edge_index,
           Wx1, bx1, Wx2, bx2, Wx3, bx3, Wx4, bx4,
           Wy1, by1, Wy2, by2, Wy3, by3, Wy4, by4):
    zeros_rpt = jnp.zeros((_RPT, _D), dtype=jnp.float32)
    xo = _branch(x_data_matrix, x_edge_index,
                 (Wx1, bx1, Wx2, bx2, Wx3, bx3, Wx4, bx4), zeros_rpt)
    yo = _branch(y_data_matrix, y_edge_index,
                 (Wy1, by1, Wy2, by2, Wy3, by3, Wy4, by4), zeros_rpt)
    return (xo, yo)
